# baseline (speedup 1.0000x reference)
import functools

import jax
import jax.numpy as jnp
from jax import lax
from jax.experimental import pallas as pl
from jax.experimental.pallas import tpu as pltpu
from jax.experimental.pallas import tpu_sc as plsc

N_NODES = 10000
N_INC = 320000
N_HE = 2500
N_HE_PAD = 2560
HE_SLICE = 160
N_NODES_PAD = 10112
NODE_SLICE = 632
NUM_CORES = 2
NUM_SUBCORES = 16
NW = NUM_CORES * NUM_SUBCORES
PER_TILE = N_INC // NW
CHUNK = 80
NCHUNK = PER_TILE // CHUNK
DEG_CHUNK = 80
DEG_NCHUNK = PER_TILE // DEG_CHUNK
DEG_RING = 5

_MESH = plsc.VectorSubcoreMesh(
    core_axis_name="c", subcore_axis_name="s",
    num_cores=NUM_CORES, num_subcores=NUM_SUBCORES)


def _degrees_body(vi_hbm, ei_hbm, ones_hbm, z_hbm, dv_out, de_out,
                  vbuf, ebuf, ones_v, *refs):
    dsem = list(refs[:DEG_RING])
    esem = list(refs[DEG_RING:2 * DEG_RING])
    dv_sh, de_sh = refs[2 * DEG_RING], refs[2 * DEG_RING + 1]
    cid = lax.axis_index("c")
    sid = lax.axis_index("s")
    wid = cid * NUM_SUBCORES + sid
    pltpu.sync_copy(z_hbm, dv_sh.at[pl.ds(sid * NODE_SLICE, NODE_SLICE)])
    pltpu.sync_copy(z_hbm.at[pl.ds(0, HE_SLICE)],
                    de_sh.at[pl.ds(sid * HE_SLICE, HE_SLICE)])
    pltpu.sync_copy(ones_hbm, ones_v)
    pltpu.sync_copy(vi_hbm.at[wid], vbuf)
    pltpu.sync_copy(ei_hbm.at[wid], ebuf)
    plsc.subcore_barrier()

    def dv_add(jj, b):
        pltpu.async_copy(ones_v, dv_sh.at[vbuf.at[jj]], dsem[b], add=True)

    def dv_wait(jj, b):
        pltpu.make_async_copy(ones_v, dv_sh.at[vbuf.at[jj]], dsem[b]).wait()

    def de_add(jj, b):
        pltpu.async_copy(ones_v, de_sh.at[ebuf.at[jj]], esem[b], add=True)

    def de_wait(jj, b):
        pltpu.make_async_copy(ones_v, de_sh.at[ebuf.at[jj]], esem[b]).wait()

    for b in range(DEG_RING):
        dv_add(b, b)
        de_add(b, b)

    @pl.loop(0, DEG_NCHUNK // DEG_RING - 1)
    def _(s):
        for b in range(DEG_RING):
            jj = s * DEG_RING + b
            dv_wait(jj, b)
            de_wait(jj, b)
            dv_add(jj + DEG_RING, b)
            de_add(jj + DEG_RING, b)

    base = DEG_NCHUNK - DEG_RING
    for b in range(DEG_RING):
        dv_wait(base + b, b)
        de_wait(base + b, b)

    plsc.subcore_barrier()
    pltpu.sync_copy(dv_sh.at[pl.ds(sid * NODE_SLICE, NODE_SLICE)],
                    dv_out.at[cid, pl.ds(sid * NODE_SLICE, NODE_SLICE)])
    pltpu.sync_copy(de_sh.at[pl.ds(sid * HE_SLICE, HE_SLICE)],
                    de_out.at[cid, pl.ds(sid * HE_SLICE, HE_SLICE)])


def _make_degrees():
    return pl.kernel(
        _degrees_body,
        out_type=(jax.ShapeDtypeStruct((NUM_CORES, N_NODES_PAD, 16), jnp.float32),
                  jax.ShapeDtypeStruct((NUM_CORES, N_HE_PAD, 16), jnp.float32)),
        mesh=_MESH,
        scratch_types=[
            pltpu.VMEM((DEG_NCHUNK, DEG_CHUNK), jnp.int32),
            pltpu.VMEM((DEG_NCHUNK, DEG_CHUNK), jnp.int32),
            pltpu.VMEM((DEG_CHUNK, 16), jnp.float32),
        ] + [pltpu.SemaphoreType.DMA for _ in range(2 * DEG_RING)] + [
            pltpu.VMEM_SHARED((N_NODES_PAD, 16), jnp.float32),
            pltpu.VMEM_SHARED((N_HE_PAD, 16), jnp.float32),
        ])


def _smooth_body(slice_rows, ring, src_hbm, ip_hbm, z_hbm, out_hbm, *refs):
    pb = list(refs[0:ring])
    rows = list(refs[ring:2 * ring])
    isem = list(refs[2 * ring:3 * ring])
    gsem = list(refs[3 * ring:4 * ring])
    ssem = list(refs[4 * ring:5 * ring])
    acc_sh = refs[5 * ring]
    cid = lax.axis_index("c")
    sid = lax.axis_index("s")
    wid = cid * NUM_SUBCORES + sid
    pltpu.sync_copy(z_hbm, acc_sh.at[pl.ds(sid * slice_rows, slice_rows)])
    plsc.subcore_barrier()

    def idx_fetch(jj, b):
        pltpu.async_copy(ip_hbm.at[wid, jj], pb[b], isem[b])

    def idx_wait(jj, b):
        pltpu.make_async_copy(ip_hbm.at[wid, jj], pb[b], isem[b]).wait()

    def gather(jj, b):
        pltpu.async_copy(src_hbm.at[pb[b].at[0]], rows[b], gsem[b])

    def gather_wait(jj, b):
        pltpu.make_async_copy(src_hbm.at[pb[b].at[0]], rows[b],
                              gsem[b]).wait()

    def scatter(jj, b):
        pltpu.async_copy(rows[b], acc_sh.at[pb[b].at[1]], ssem[b], add=True)

    def scatter_wait(jj, b):
        pltpu.make_async_copy(rows[b], acc_sh.at[pb[b].at[1]],
                              ssem[b]).wait()

    for b in range(ring):
        idx_fetch(b, b)
    nmain = NCHUNK // ring - 1

    @pl.loop(0, nmain)
    def _(s):
        for b in range(ring):
            jj = s * ring + b
            idx_wait(jj, b)
            gather(jj, b)
            gather_wait(jj, b)
            scatter(jj, b)
            scatter_wait(jj, b)
            idx_fetch(jj + ring, b)

    base = nmain * ring
    leftover = NCHUNK - base - ring
    for b in range(ring):
        jj = base + b
        idx_wait(jj, b)
        gather(jj, b)
        gather_wait(jj, b)
        scatter(jj, b)
        scatter_wait(jj, b)
        if b < leftover:
            idx_fetch(base + ring + b, b)
    for b in range(leftover):
        jj = base + ring + b
        idx_wait(jj, b)
        gather(jj, b)
        gather_wait(jj, b)
        scatter(jj, b)
        scatter_wait(jj, b)

    plsc.subcore_barrier()
    pltpu.sync_copy(acc_sh.at[pl.ds(sid * slice_rows, slice_rows)],
                    out_hbm.at[cid, pl.ds(sid * slice_rows, slice_rows)])


def _make_smooth(c, swap, ring):
    n_acc, slice_rows = ((N_NODES_PAD, NODE_SLICE) if swap
                         else (N_HE_PAD, HE_SLICE))
    return pl.kernel(
        functools.partial(_smooth_body, slice_rows, ring),
        out_type=jax.ShapeDtypeStruct((NUM_CORES, n_acc, c), jnp.float32),
        mesh=_MESH,
        scratch_types=(
            [pltpu.VMEM((2, CHUNK), jnp.int32) for _ in range(ring)]
            + [pltpu.VMEM((CHUNK, c), jnp.float32) for _ in range(ring)]
            + [pltpu.SemaphoreType.DMA for _ in range(3 * ring)]
            + [pltpu.VMEM_SHARED((n_acc, c), jnp.float32)]
        ))


V2E_RING = 8
E2V_RING = 4


def _make_v2e(c):
    return _make_smooth(c, swap=False, ring=V2E_RING)


def _make_e2v(c):
    return _make_smooth(c, swap=True, ring=E2V_RING)


def _dv_isqrt(dvp):
    dv = dvp[0, :N_NODES, :1] + dvp[1, :N_NODES, :1]
    return jnp.where(dv > 0, lax.rsqrt(jnp.maximum(dv, 1e-12)), 0.0)


def _theta0_body(x_ref, w_ref, b_ref, dvp_ref, shift_ref, o_ref):
    x = x_ref[...] + shift_ref[0, 0]
    h = jnp.dot(x, w_ref[...], preferred_element_type=jnp.float32,
                precision=lax.Precision.HIGHEST) + b_ref[...]
    o_ref[...] = h * _dv_isqrt(dvp_ref)


def _scale_he_body(hep_ref, dep_ref, o_ref):
    he = hep_ref[0] + hep_ref[1]
    de = dep_ref[0, :, :1] + dep_ref[1, :, :1]
    dei = jnp.where(de > 0, 1.0 / jnp.maximum(de, 1e-12), 0.0)
    o_ref[...] = he * dei


def _theta1_body(np_ref, dvp_ref, w_ref, b_ref, o_ref):
    dvis = _dv_isqrt(dvp_ref)
    sm = (np_ref[0, :N_NODES] + np_ref[1, :N_NODES]) * dvis
    h = jnp.maximum(sm, 0.0)
    r = (jnp.dot(h, w_ref[...], preferred_element_type=jnp.float32,
                 precision=lax.Precision.HIGHEST) + b_ref[...]) * dvis
    o_ref[...] = jnp.concatenate([r, jnp.zeros_like(r)], axis=1)


def _final_body(c_cls, np_ref, dvp_ref, o_ref):
    sm = ((np_ref[0, :N_NODES, :c_cls] + np_ref[1, :N_NODES, :c_cls])
          * _dv_isqrt(dvp_ref))
    o_ref[...] = jax.nn.sigmoid(sm)


def _tc_call(body, out_shape):
    return pl.pallas_call(body, out_shape=out_shape)


def kernel(X, W0, b0, W1, b1, v_idx, e_idx, num_hyperedges):
    c_in = X.shape[1]
    c_hid = W0.shape[1]
    c_cls = W1.shape[1]
    f32 = jnp.float32

    vi80 = v_idx.reshape(NW, NCHUNK, CHUNK)
    ei80 = e_idx.reshape(NW, NCHUNK, CHUNK)
    pair_ve = jnp.stack([vi80, ei80], axis=2)
    pair_ev = jnp.stack([ei80, vi80], axis=2)
    ones16 = jnp.ones((DEG_CHUNK, 16), f32)
    z16 = jnp.zeros((NODE_SLICE, 16), f32)
    z_he_h = jnp.zeros((HE_SLICE, c_hid), f32)
    z_nd_h = jnp.zeros((NODE_SLICE, c_hid), f32)
    shift = (jnp.asarray(num_hyperedges) - N_HE).astype(f32).reshape(1, 1)

    dvp, dep = _make_degrees()(vi80, ei80, ones16, z16)

    h0s = _tc_call(_theta0_body,
                   jax.ShapeDtypeStruct((N_NODES, c_hid), f32))(
        X, W0, b0.reshape(1, c_hid), dvp, shift)

    hep = _make_v2e(c_hid)(h0s, pair_ve, z_he_h)
    hes = _tc_call(_scale_he_body,
                   jax.ShapeDtypeStruct((N_HE_PAD, c_hid), f32))(hep, dep)
    ndp = _make_e2v(c_hid)(hes, pair_ev, z_nd_h)

    h1s = _tc_call(_theta1_body,
                   jax.ShapeDtypeStruct((N_NODES, 2 * c_cls), f32))(
        ndp, dvp, W1, b1.reshape(1, c_cls))

    hep2 = _make_v2e(c_hid)(h1s, pair_ve, z_he_h)
    hes2 = _tc_call(_scale_he_body,
                    jax.ShapeDtypeStruct((N_HE_PAD, c_hid), f32))(hep2, dep)
    ndp2 = _make_e2v(c_hid)(hes2, pair_ev, z_nd_h)

    out = _tc_call(functools.partial(_final_body, c_cls),
                   jax.ShapeDtypeStruct((N_NODES, c_cls), f32))(ndp2, dvp)
    return out


# speedup vs baseline: 1.7313x; 1.7313x over previous
import functools

import jax
import jax.numpy as jnp
from jax import lax
from jax.experimental import pallas as pl
from jax.experimental.pallas import tpu as pltpu
from jax.experimental.pallas import tpu_sc as plsc

N_NODES = 10000
N_INC = 320000
N_HE = 2500
N_HE_PAD = 2560
HE_SLICE = 160
N_NODES_PAD = 10112
NODE_SLICE = 632
NUM_CORES = 2
NUM_SUBCORES = 16
NW = NUM_CORES * NUM_SUBCORES
PER_TILE = N_INC // NW
CHUNK = 80
NCHUNK = PER_TILE // CHUNK
V2E_RING = 5
E2V_RING = 2
DEG_RING = 5

_MESH = plsc.VectorSubcoreMesh(
    core_axis_name="c", subcore_axis_name="s",
    num_cores=NUM_CORES, num_subcores=NUM_SUBCORES)


def _degrees_body(vi_hbm, ei_hbm, ones_hbm, z_hbm, dv_out, de_out,
                  vbuf, ebuf, ones_v, *refs):
    dsem = list(refs[:DEG_RING])
    esem = list(refs[DEG_RING:2 * DEG_RING])
    dv_sh, de_sh = refs[2 * DEG_RING], refs[2 * DEG_RING + 1]
    cid = lax.axis_index("c")
    sid = lax.axis_index("s")
    wid = cid * NUM_SUBCORES + sid
    pltpu.sync_copy(z_hbm, dv_sh.at[pl.ds(sid * NODE_SLICE, NODE_SLICE)])
    pltpu.sync_copy(z_hbm.at[pl.ds(0, HE_SLICE)],
                    de_sh.at[pl.ds(sid * HE_SLICE, HE_SLICE)])
    pltpu.sync_copy(ones_hbm, ones_v)
    pltpu.sync_copy(vi_hbm.at[wid], vbuf)
    pltpu.sync_copy(ei_hbm.at[wid], ebuf)
    plsc.subcore_barrier()

    def dv_add(jj, b):
        pltpu.async_copy(ones_v, dv_sh.at[vbuf.at[jj]], dsem[b], add=True)

    def dv_wait(jj, b):
        pltpu.make_async_copy(ones_v, dv_sh.at[vbuf.at[jj]], dsem[b]).wait()

    def de_add(jj, b):
        pltpu.async_copy(ones_v, de_sh.at[ebuf.at[jj]], esem[b], add=True)

    def de_wait(jj, b):
        pltpu.make_async_copy(ones_v, de_sh.at[ebuf.at[jj]], esem[b]).wait()

    for b in range(DEG_RING):
        dv_add(b, b)
        de_add(b, b)

    @pl.loop(0, NCHUNK // DEG_RING - 1)
    def _(s):
        for b in range(DEG_RING):
            jj = s * DEG_RING + b
            dv_wait(jj, b)
            de_wait(jj, b)
            dv_add(jj + DEG_RING, b)
            de_add(jj + DEG_RING, b)

    base = NCHUNK - DEG_RING
    for b in range(DEG_RING):
        dv_wait(base + b, b)
        de_wait(base + b, b)

    plsc.subcore_barrier()
    pltpu.sync_copy(dv_sh.at[pl.ds(sid * NODE_SLICE, NODE_SLICE)],
                    dv_out.at[cid, pl.ds(sid * NODE_SLICE, NODE_SLICE)])
    pltpu.sync_copy(de_sh.at[pl.ds(sid * HE_SLICE, HE_SLICE)],
                    de_out.at[cid, pl.ds(sid * HE_SLICE, HE_SLICE)])


def _make_degrees():
    return pl.kernel(
        _degrees_body,
        out_type=(jax.ShapeDtypeStruct((NUM_CORES, N_NODES_PAD, 16), jnp.float32),
                  jax.ShapeDtypeStruct((NUM_CORES, N_HE_PAD, 16), jnp.float32)),
        mesh=_MESH,
        scratch_types=[
            pltpu.VMEM((NCHUNK, CHUNK), jnp.int32),
            pltpu.VMEM((NCHUNK, CHUNK), jnp.int32),
            pltpu.VMEM((CHUNK, 16), jnp.float32),
        ] + [pltpu.SemaphoreType.DMA for _ in range(2 * DEG_RING)] + [
            pltpu.VMEM_SHARED((N_NODES_PAD, 16), jnp.float32),
            pltpu.VMEM_SHARED((N_HE_PAD, 16), jnp.float32),
        ])


def _v2e_body(c, src_hbm, vi_hbm, ei_hbm, z_hbm, out_hbm, *refs):
    ring = V2E_RING
    vbuf, ebuf = refs[0], refs[1]
    rows = list(refs[2:2 + ring])
    gsem = list(refs[2 + ring:2 + 2 * ring])
    ssem = refs[2 + 2 * ring]
    acc_sh = refs[3 + 2 * ring]
    cid = lax.axis_index("c")
    sid = lax.axis_index("s")
    wid = cid * NUM_SUBCORES + sid
    pltpu.sync_copy(z_hbm, acc_sh.at[pl.ds(sid * HE_SLICE, HE_SLICE)])
    pltpu.sync_copy(vi_hbm.at[wid], vbuf)
    pltpu.sync_copy(ei_hbm.at[wid], ebuf)
    plsc.subcore_barrier()

    def gather(jj, b):
        pltpu.async_copy(src_hbm.at[vbuf.at[jj]], rows[b], gsem[b])

    def gather_wait(jj, b):
        pltpu.make_async_copy(src_hbm.at[vbuf.at[jj]], rows[b],
                              gsem[b]).wait()

    def scatter(jj, b):
        pltpu.async_copy(rows[b], acc_sh.at[ebuf.at[jj]], ssem, add=True)
        pltpu.make_async_copy(rows[b], acc_sh.at[ebuf.at[jj]], ssem).wait()

    for b in range(ring):
        gather(b, b)

    @pl.loop(0, NCHUNK // ring - 1)
    def _(s):
        for b in range(ring):
            jj = s * ring + b
            gather_wait(jj, b)
            scatter(jj, b)
            gather(jj + ring, b)

    base = NCHUNK - ring
    for b in range(ring):
        gather_wait(base + b, b)
        scatter(base + b, b)

    plsc.subcore_barrier()
    pltpu.sync_copy(acc_sh.at[pl.ds(sid * HE_SLICE, HE_SLICE)],
                    out_hbm.at[cid, pl.ds(sid * HE_SLICE, HE_SLICE)])


def _make_v2e(c):
    return pl.kernel(
        functools.partial(_v2e_body, c),
        out_type=jax.ShapeDtypeStruct((NUM_CORES, N_HE_PAD, c), jnp.float32),
        mesh=_MESH,
        scratch_types=(
            [pltpu.VMEM((NCHUNK, CHUNK), jnp.int32),
             pltpu.VMEM((NCHUNK, CHUNK), jnp.int32)]
            + [pltpu.VMEM((CHUNK, c), jnp.float32) for _ in range(V2E_RING)]
            + [pltpu.SemaphoreType.DMA for _ in range(V2E_RING + 1)]
            + [pltpu.VMEM_SHARED((N_HE_PAD, c), jnp.float32)]
        ))


def _e2v_body(c, src_hbm, vi_hbm, ei_hbm, z_hbm, out_hbm, *refs):
    ring = E2V_RING
    vbuf = refs[0]
    epb = [list(refs[1:1 + ring]), list(refs[1 + ring:1 + 2 * ring])]
    rows = list(refs[1 + 2 * ring:1 + 3 * ring])
    isem = [list(refs[1 + 3 * ring:1 + 4 * ring]),
            list(refs[1 + 4 * ring:1 + 5 * ring])]
    gsem = list(refs[1 + 5 * ring:1 + 6 * ring])
    ssem = refs[1 + 6 * ring]
    acc_sh = refs[2 + 6 * ring]
    cid = lax.axis_index("c")
    sid = lax.axis_index("s")
    wid = cid * NUM_SUBCORES + sid
    start = jnp.minimum(sid * NODE_SLICE, N_NODES - NODE_SLICE)
    pltpu.sync_copy(z_hbm, acc_sh.at[pl.ds(start, NODE_SLICE)])
    pltpu.sync_copy(vi_hbm.at[wid], vbuf)
    plsc.subcore_barrier()

    def idx_fetch(jj, b, p):
        pltpu.async_copy(ei_hbm.at[wid, pl.ds(jj, 1)], epb[p][b],
                         isem[p][b])

    def idx_wait(jj, b, p):
        pltpu.make_async_copy(ei_hbm.at[wid, pl.ds(jj, 1)], epb[p][b],
                              isem[p][b]).wait()

    def gather(jj, b, p):
        pltpu.async_copy(src_hbm.at[epb[p][b].at[0]], rows[b], gsem[b])

    def gather_wait(jj, b, p):
        pltpu.make_async_copy(src_hbm.at[epb[p][b].at[0]], rows[b],
                              gsem[b]).wait()

    def scatter(jj, b):
        pltpu.async_copy(rows[b], acc_sh.at[vbuf.at[jj]], ssem, add=True)
        pltpu.make_async_copy(rows[b], acc_sh.at[vbuf.at[jj]], ssem).wait()

    def visit(jj, b, p, guard=False):
        gather_wait(jj, b, p)
        scatter(jj, b)
        njj, fjj = jj + ring, jj + 2 * ring
        if not guard or njj < NCHUNK:
            idx_wait(njj, b, 1 - p)
            gather(njj, b, 1 - p)
        if not guard or fjj < NCHUNK:
            idx_fetch(fjj, b, p)

    ncyc_full = NCHUNK // ring
    ncyc = -(-NCHUNK // ring)
    npair = (ncyc_full - 2) // 2

    for b in range(ring):
        idx_fetch(b, b, 0)
    for b in range(ring):
        idx_wait(b, b, 0)
        gather(b, b, 0)
        idx_fetch(ring + b, b, 1)

    @pl.loop(0, npair)
    def _(s2):
        for sp in (0, 1):
            for b in range(ring):
                visit((s2 * 2 + sp) * ring + b, b, sp)

    for s in range(2 * npair, ncyc):
        for b in range(ring):
            jj = s * ring + b
            if jj < NCHUNK:
                visit(jj, b, s & 1, guard=True)

    plsc.subcore_barrier()
    pltpu.sync_copy(acc_sh.at[pl.ds(start, NODE_SLICE)],
                    out_hbm.at[cid, pl.ds(start, NODE_SLICE)])


def _make_e2v(c):
    return pl.kernel(
        functools.partial(_e2v_body, c),
        out_type=jax.ShapeDtypeStruct((NUM_CORES, N_NODES, c), jnp.float32),
        mesh=_MESH,
        scratch_types=(
            [pltpu.VMEM((NCHUNK, CHUNK), jnp.int32)]
            + [pltpu.VMEM((1, CHUNK), jnp.int32) for _ in range(2 * E2V_RING)]
            + [pltpu.VMEM((CHUNK, c), jnp.float32) for _ in range(E2V_RING)]
            + [pltpu.SemaphoreType.DMA for _ in range(3 * E2V_RING + 1)]
            + [pltpu.VMEM_SHARED((N_NODES, c), jnp.float32)]
        ))


def _dv_isqrt(dvp):
    dv = dvp[0, :N_NODES, :1] + dvp[1, :N_NODES, :1]
    return jnp.where(dv > 0, lax.rsqrt(jnp.maximum(dv, 1e-12)), 0.0)


def _theta0_body(x_ref, w_ref, b_ref, dvp_ref, shift_ref, o_ref):
    x = x_ref[...] + shift_ref[0, 0]
    h = jnp.dot(x, w_ref[...], preferred_element_type=jnp.float32,
                precision=lax.Precision.HIGHEST) + b_ref[...]
    o_ref[...] = h * _dv_isqrt(dvp_ref)


def _scale_he_body(hep_ref, dep_ref, o_ref):
    he = hep_ref[0] + hep_ref[1]
    de = dep_ref[0, :, :1] + dep_ref[1, :, :1]
    dei = jnp.where(de > 0, 1.0 / jnp.maximum(de, 1e-12), 0.0)
    o_ref[...] = he * dei


def _theta1_body(np_ref, dvp_ref, w_ref, b_ref, o_ref):
    dvis = _dv_isqrt(dvp_ref)
    sm = (np_ref[0] + np_ref[1]) * dvis
    h = jnp.maximum(sm, 0.0)
    r = (jnp.dot(h, w_ref[...], preferred_element_type=jnp.float32,
                 precision=lax.Precision.HIGHEST) + b_ref[...]) * dvis
    o_ref[...] = jnp.concatenate([r, jnp.zeros_like(r)], axis=1)


def _final_body(c_cls, np_ref, dvp_ref, o_ref):
    sm = ((np_ref[0, :, :c_cls] + np_ref[1, :, :c_cls])
          * _dv_isqrt(dvp_ref))
    o_ref[...] = jax.nn.sigmoid(sm)


def _tc_call(body, out_shape):
    return pl.pallas_call(body, out_shape=out_shape)


def kernel(X, W0, b0, W1, b1, v_idx, e_idx, num_hyperedges):
    c_hid = W0.shape[1]
    c_cls = W1.shape[1]
    f32 = jnp.float32

    vi3 = v_idx.reshape(NW, NCHUNK, CHUNK)
    ei3 = e_idx.reshape(NW, NCHUNK, CHUNK)
    ones16 = jnp.ones((CHUNK, 16), f32)
    z16 = jnp.zeros((NODE_SLICE, 16), f32)
    z_he = jnp.zeros((HE_SLICE, c_hid), f32)
    z_nd = jnp.zeros((NODE_SLICE, c_hid), f32)
    shift = (jnp.asarray(num_hyperedges) - N_HE).astype(f32).reshape(1, 1)

    dvp, dep = _make_degrees()(vi3, ei3, ones16, z16)

    h0s = _tc_call(_theta0_body,
                   jax.ShapeDtypeStruct((N_NODES, c_hid), f32))(
        X, W0, b0.reshape(1, c_hid), dvp, shift)

    hep = _make_v2e(c_hid)(h0s, vi3, ei3, z_he)
    hes = _tc_call(_scale_he_body,
                   jax.ShapeDtypeStruct((N_HE_PAD, c_hid), f32))(hep, dep)
    ndp = _make_e2v(c_hid)(hes, vi3, ei3, z_nd)

    h1s = _tc_call(_theta1_body,
                   jax.ShapeDtypeStruct((N_NODES, 2 * c_cls), f32))(
        ndp, dvp, W1, b1.reshape(1, c_cls))

    hep2 = _make_v2e(c_hid)(h1s, vi3, ei3, z_he)
    hes2 = _tc_call(_scale_he_body,
                    jax.ShapeDtypeStruct((N_HE_PAD, c_hid), f32))(hep2, dep)
    ndp2 = _make_e2v(c_hid)(hes2, vi3, ei3, z_nd)

    out = _tc_call(functools.partial(_final_body, c_cls),
                   jax.ShapeDtypeStruct((N_NODES, c_cls), f32))(ndp2, dvp)
    return out
